# Initial kernel scaffold; baseline (speedup 1.0000x reference)
import jax
import jax.numpy as jnp
from jax.experimental import pallas as pl


def kernel(pred, target, x, pos, edge_index, edge_attr):
    raise NotImplementedError("write your pallas kernel here")


# speedup vs baseline: 21.0790x; 21.0790x over previous
import functools

import jax
import jax.numpy as jnp
from jax import lax
from jax.experimental import pallas as pl
from jax.experimental.pallas import tpu as pltpu
from jax.experimental.pallas import tpu_sc as plsc

_N = 100000
_E = 6400000
_NC = 2
_NS = 16
_NW = _NC * _NS
_EPW = _E // _NW
_K = 2000
_NCHUNK = _EPW // _K
_G = _K // 16
_RPT = _N // _NS
_ASTRIPE = 6256

_R = 2000
_GRID = _N // _R

_mesh = plsc.VectorSubcoreMesh(core_axis_name="c", subcore_axis_name="s")


def _splat_i32(c):
    return jnp.full((16,), c, jnp.int32)


def _edge_ab_body(tbl_hbm, rows_hbm, cols_hbm, ea_hbm, z8_hbm, out_hbm,
                  rows_v, cols_v, arow_v, acol_v, ea_v, contrib_v, acc_sh,
                  sem):
    cid = lax.axis_index("c")
    sid = lax.axis_index("s")
    wid = cid * _NS + sid

    pltpu.sync_copy(z8_hbm, acc_sh.at[pl.ds(sid * _RPT, _RPT)])
    pltpu.sync_copy(z8_hbm.at[pl.ds(0, _K)], contrib_v)
    plsc.subcore_barrier()

    lane = lax.iota(jnp.int32, 16)

    def chunk(i, carry):
        base = wid * _EPW + i * _K
        c1 = pltpu.async_copy(rows_hbm.at[pl.ds(base, _K)], rows_v, sem)
        c2 = pltpu.async_copy(cols_hbm.at[pl.ds(base, _K)], cols_v, sem)
        c3 = pltpu.async_copy(ea_hbm.at[pl.ds(base, _K)], ea_v, sem)
        c1.wait()
        c2.wait()
        c3.wait()
        g1 = pltpu.async_copy(tbl_hbm.at[rows_v], arow_v, sem)
        g2 = pltpu.async_copy(tbl_hbm.at[cols_v], acol_v, sem)
        g1.wait()
        g2.wait()

        def grp(g, carry2):
            idx = g * 16 + lane

            def ld(ref, c):
                return plsc.load_gather(ref, [idx, _splat_i32(c)])

            phir = ld(arow_v, 0)
            pxr = ld(arow_v, 1)
            pyr = ld(arow_v, 2)
            pzr = ld(arow_v, 3)
            phic = ld(acol_v, 0)
            pxc = ld(acol_v, 1)
            pyc = ld(acol_v, 2)
            pzc = ld(acol_v, 3)
            ax = ld(ea_v, 0)
            ay = ld(ea_v, 1)
            az = ld(ea_v, 2)

            fd = phic - phir
            sa = ax * ax + ay * ay + az * az
            wa = fd / jnp.maximum(sa, 1e-16)
            rx = pxc - pxr
            ry = pyc - pyr
            rz = pzc - pzr
            sb = rx * rx + ry * ry + rz * rz
            wb = fd / jnp.maximum(sb, 1e-16)

            def st(c, val):
                plsc.store_scatter(contrib_v, [idx, _splat_i32(c)], val)

            st(0, wa * ax)
            st(1, wa * ay)
            st(2, wa * az)
            st(3, wb * rx)
            st(4, wb * ry)
            st(5, wb * rz)
            st(6, jnp.full((16,), 1.0, jnp.float32))
            return carry2

        lax.fori_loop(0, _G, grp, 0)
        pltpu.sync_copy(contrib_v, acc_sh.at[rows_v], add=True)
        return carry

    lax.fori_loop(0, _NCHUNK, chunk, 0)
    plsc.subcore_barrier()
    pltpu.sync_copy(acc_sh.at[pl.ds(sid * _RPT, _RPT)],
                    out_hbm.at[pl.ds(cid * _N + sid * _RPT, _RPT)])


_edge_ab = functools.partial(
    pl.kernel,
    out_type=jax.ShapeDtypeStruct((2 * _N, 8), jnp.float32),
    mesh=_mesh,
    compiler_params=pltpu.CompilerParams(use_tc_tiling_on_sc=False, needs_layout_passes=False),
    scratch_types=[
        pltpu.VMEM((_K,), jnp.int32),
        pltpu.VMEM((_K,), jnp.int32),
        pltpu.VMEM((_K, 8), jnp.float32),
        pltpu.VMEM((_K, 8), jnp.float32),
        pltpu.VMEM((_K, 4), jnp.float32),
        pltpu.VMEM((_K, 8), jnp.float32),
        pltpu.VMEM_SHARED((_N, 8), jnp.float32),
        pltpu.SemaphoreType.DMA,
    ],
)(_edge_ab_body)


def _edge_div_body(tbl_hbm, rows_hbm, cols_hbm, z1_hbm, out_hbm,
                   rows_v, cols_v, arow_v, acol_v, contrib_v, acc_sh, sem):
    cid = lax.axis_index("c")
    sid = lax.axis_index("s")
    wid = cid * _NS + sid

    astart = pl.multiple_of(((sid * _RPT) >> 3) << 3, 8)
    pltpu.sync_copy(z1_hbm.at[pl.ds(0, _ASTRIPE)],
                    acc_sh.at[pl.ds(astart, _ASTRIPE)])
    plsc.subcore_barrier()

    lane = lax.iota(jnp.int32, 16)

    def chunk(i, carry):
        base = wid * _EPW + i * _K
        c1 = pltpu.async_copy(rows_hbm.at[pl.ds(base, _K)], rows_v, sem)
        c2 = pltpu.async_copy(cols_hbm.at[pl.ds(base, _K)], cols_v, sem)
        c1.wait()
        c2.wait()
        g1 = pltpu.async_copy(tbl_hbm.at[rows_v], arow_v, sem)
        g2 = pltpu.async_copy(tbl_hbm.at[cols_v], acol_v, sem)
        g1.wait()
        g2.wait()

        def grp(g, carry2):
            idx = g * 16 + lane

            def ld(ref, c):
                return plsc.load_gather(ref, [idx, _splat_i32(c)])

            nxr = ld(arow_v, 0)
            nyr = ld(arow_v, 1)
            nzr = ld(arow_v, 2)
            pxr = ld(arow_v, 3)
            pyr = ld(arow_v, 4)
            pzr = ld(arow_v, 5)
            nxc = ld(acol_v, 0)
            nyc = ld(acol_v, 1)
            nzc = ld(acol_v, 2)
            pxc = ld(acol_v, 3)
            pyc = ld(acol_v, 4)
            pzc = ld(acol_v, 5)

            rx = pxc - pxr
            ry = pyc - pyr
            rz = pzc - pzr
            sb = rx * rx + ry * ry + rz * rz
            ii = plsc.bitcast(sb, jnp.int32)
            ii = jnp.int32(0x5F3759DF) - (ii >> 1)
            y = plsc.bitcast(ii, jnp.float32)
            y = y * (1.5 - 0.5 * sb * y * y)
            y = y * (1.5 - 0.5 * sb * y * y)
            dist = sb * y
            den = jnp.maximum(dist, 1e-8) + 1e-8
            num = ((nxc - nxr) * rx + (nyc - nyr) * ry + (nzc - nzr) * rz)
            contrib_v[pl.ds(g * 16, 16)] = num / den
            return carry2

        lax.fori_loop(0, _G, grp, 0)
        pltpu.sync_copy(contrib_v, acc_sh.at[rows_v], add=True)
        return carry

    lax.fori_loop(0, _NCHUNK, chunk, 0)
    plsc.subcore_barrier()
    astart2 = pl.multiple_of(((sid * _RPT) >> 3) << 3, 8)
    pltpu.sync_copy(acc_sh.at[pl.ds(astart2, _ASTRIPE)],
                    out_hbm.at[pl.ds(cid * _N + astart2, _ASTRIPE)])


_edge_div = functools.partial(
    pl.kernel,
    out_type=jax.ShapeDtypeStruct((2 * _N,), jnp.float32),
    mesh=_mesh,
    compiler_params=pltpu.CompilerParams(use_tc_tiling_on_sc=False, needs_layout_passes=False),
    scratch_types=[
        pltpu.VMEM((_K,), jnp.int32),
        pltpu.VMEM((_K,), jnp.int32),
        pltpu.VMEM((_K, 8), jnp.float32),
        pltpu.VMEM((_K, 8), jnp.float32),
        pltpu.VMEM((_K,), jnp.float32),
        pltpu.VMEM_SHARED((_N,), jnp.float32),
        pltpu.SemaphoreType.DMA,
    ],
)(_edge_div_body)


def _node_body(a0, a1, tbl, tblc, dinv, scal):
    i = pl.program_id(0)
    a = a0[...] + a1[...]
    deg = jnp.maximum(a[:, 6:7], 1.0)
    ga = a[:, 0:3] / deg
    gb = a[:, 3:6] / deg
    gan = jnp.sqrt(jnp.sum(ga * ga, axis=1, keepdims=True))
    gbn = jnp.maximum(jnp.sqrt(jnp.sum(gb * gb, axis=1, keepdims=True)), 1e-8)
    normal = gb / gbn
    phi = tbl[:, 0:1]
    pos = tbl[:, 1:4]
    phic = tbl[:, 4:5]
    tblc[...] = jnp.concatenate(
        [normal, pos, jnp.zeros_like(normal[:, 0:2])], axis=1)
    dinv[...] = 1.0 / deg
    mask = jnp.abs(phi) < 0.3
    isum = jnp.sum(jnp.where(mask, (gan - 1.0) ** 2, 0.0))
    icnt = jnp.sum(mask.astype(jnp.float32))
    vc = jnp.sum((phic > 0).astype(jnp.float32))
    vn = jnp.sum((phi > 0).astype(jnp.float32))
    lanes = lax.broadcasted_iota(jnp.int32, (1, 128), 1)
    part = (jnp.where(lanes == 0, isum, 0.0)
            + jnp.where(lanes == 1, icnt, 0.0)
            + jnp.where(lanes == 2, vc, 0.0)
            + jnp.where(lanes == 3, vn, 0.0))

    @pl.when(i == 0)
    def _():
        scal[...] = jnp.zeros_like(scal)

    scal[...] += part


def _loss_body(c0, c1, dinv, scal, tot):
    i = pl.program_id(0)
    curv = (c0[...] + c1[...]) * dinv[...]
    part = jnp.sum(curv * curv).reshape(1, 1)

    @pl.when(i == 0)
    def _():
        tot[...] = jnp.zeros_like(tot)

    tot[...] += part

    @pl.when(i == pl.num_programs(0) - 1)
    def _():
        s = tot[...][0, 0]
        sv = scal[...]
        isum = sv[0, 0]
        icnt = sv[0, 1]
        vc = sv[0, 2]
        vn = sv[0, 3]
        interface = jnp.where(icnt > 0, isum / jnp.maximum(icnt, 1.0), 0.0)
        vol = ((vn - vc) / (vc + 1e-8)) ** 2
        total = interface + 0.1 * (s / _N) + 0.01 * vol
        tot[...] = total.reshape(1, 1)


def kernel(pred, target, x, pos, edge_index, edge_attr):
    phi_c = x[:, 8]
    phi = phi_c + pred[:, 5]
    zeros3 = jnp.zeros((_N, 3), jnp.float32)
    tbl1 = jnp.concatenate(
        [phi[:, None], pos, phi_c[:, None], zeros3], axis=1)
    rows = edge_index[0]
    cols = edge_index[1]
    z8 = jnp.zeros((_RPT, 8), jnp.float32)
    z1 = jnp.zeros((_ASTRIPE,), jnp.float32)

    acc2 = _edge_ab(tbl1, rows, cols, edge_attr, z8)

    tblc, dinv, scal = pl.pallas_call(
        _node_body,
        grid=(_GRID,),
        in_specs=[
            pl.BlockSpec((_R, 8), lambda i: (i, 0)),
            pl.BlockSpec((_R, 8), lambda i: (i + _GRID, 0)),
            pl.BlockSpec((_R, 8), lambda i: (i, 0)),
        ],
        out_specs=[
            pl.BlockSpec((_R, 8), lambda i: (i, 0)),
            pl.BlockSpec((_R, 1), lambda i: (i, 0)),
            pl.BlockSpec((1, 128), lambda i: (0, 0)),
        ],
        out_shape=[
            jax.ShapeDtypeStruct((_N, 8), jnp.float32),
            jax.ShapeDtypeStruct((_N, 1), jnp.float32),
            jax.ShapeDtypeStruct((1, 128), jnp.float32),
        ],
    )(acc2, acc2, tbl1)

    curv2 = _edge_div(tblc, rows, cols, z1)
    curv2 = curv2.reshape(2 * _N, 1)

    tot = pl.pallas_call(
        _loss_body,
        grid=(_GRID,),
        in_specs=[
            pl.BlockSpec((_R, 1), lambda i: (i, 0)),
            pl.BlockSpec((_R, 1), lambda i: (i + _GRID, 0)),
            pl.BlockSpec((_R, 1), lambda i: (i, 0)),
            pl.BlockSpec((1, 128), lambda i: (0, 0)),
        ],
        out_specs=pl.BlockSpec((1, 1), lambda i: (0, 0)),
        out_shape=jax.ShapeDtypeStruct((1, 1), jnp.float32),
    )(curv2, curv2, dinv, scal)

    return tot[0, 0]


# speedup vs baseline: 24.0122x; 1.1391x over previous
import functools

import jax
import jax.numpy as jnp
from jax import lax
from jax.experimental import pallas as pl
from jax.experimental.pallas import tpu as pltpu
from jax.experimental.pallas import tpu_sc as plsc

_N = 100000
_E = 6400000
_NC = 2
_NS = 16
_NW = _NC * _NS
_EPW = _E // _NW
_K = 2000
_NCHUNK = _EPW // _K
_G = _K // 16
_RPT = _N // _NS
_ASTRIPE = 6256

_R = 2000
_GRID = _N // _R

_mesh = plsc.VectorSubcoreMesh(core_axis_name="c", subcore_axis_name="s")


def _splat_i32(c):
    return jnp.full((16,), c, jnp.int32)


def _edge_ab_body(tbl_hbm, ei_hbm, ea_hbm, z8_hbm, out_hbm,
                  rows_v, cols_v, arow_v, acol_v, ea_v, contrib_v, acc_sh,
                  sem):
    cid = lax.axis_index("c")
    sid = lax.axis_index("s")
    wid = cid * _NS + sid

    pltpu.sync_copy(z8_hbm, acc_sh.at[pl.ds(sid * _RPT, _RPT)])
    pltpu.sync_copy(z8_hbm.at[pl.ds(0, _K)], contrib_v)
    plsc.subcore_barrier()

    lane = lax.iota(jnp.int32, 16)

    def chunk(i, carry):
        base = wid * _EPW + i * _K
        c1 = pltpu.async_copy(ei_hbm.at[pl.ds(base, _K)], rows_v, sem)
        c2 = pltpu.async_copy(ei_hbm.at[pl.ds(_E + base, _K)], cols_v, sem)
        c3 = pltpu.async_copy(ea_hbm.at[pl.ds(4 * base, 4 * _K)], ea_v, sem)
        c1.wait()
        c2.wait()
        c3.wait()
        g1 = pltpu.async_copy(tbl_hbm.at[rows_v], arow_v, sem)
        g2 = pltpu.async_copy(tbl_hbm.at[cols_v], acol_v, sem)
        g1.wait()
        g2.wait()

        def grp(g, carry2):
            idx = g * 16 + lane

            def ld(ref, c):
                return plsc.load_gather(ref, [idx, _splat_i32(c)])

            phir = ld(arow_v, 0)
            pxr = ld(arow_v, 1)
            pyr = ld(arow_v, 2)
            pzr = ld(arow_v, 3)
            phic = ld(acol_v, 0)
            pxc = ld(acol_v, 1)
            pyc = ld(acol_v, 2)
            pzc = ld(acol_v, 3)
            idx4 = idx * 4

            def lda(c):
                return plsc.load_gather(ea_v, [idx4 + c])

            ax = lda(0)
            ay = lda(1)
            az = lda(2)

            fd = phic - phir
            sa = ax * ax + ay * ay + az * az
            wa = fd / jnp.maximum(sa, 1e-16)
            rx = pxc - pxr
            ry = pyc - pyr
            rz = pzc - pzr
            sb = rx * rx + ry * ry + rz * rz
            wb = fd / jnp.maximum(sb, 1e-16)

            def st(c, val):
                plsc.store_scatter(contrib_v, [idx, _splat_i32(c)], val)

            st(0, wa * ax)
            st(1, wa * ay)
            st(2, wa * az)
            st(3, wb * rx)
            st(4, wb * ry)
            st(5, wb * rz)
            st(6, jnp.full((16,), 1.0, jnp.float32))
            return carry2

        lax.fori_loop(0, _G, grp, 0)
        pltpu.sync_copy(contrib_v, acc_sh.at[rows_v], add=True)
        return carry

    lax.fori_loop(0, _NCHUNK, chunk, 0)
    plsc.subcore_barrier()
    pltpu.sync_copy(acc_sh.at[pl.ds(sid * _RPT, _RPT)],
                    out_hbm.at[pl.ds(cid * _N + sid * _RPT, _RPT)])


_edge_ab = functools.partial(
    pl.kernel,
    out_type=jax.ShapeDtypeStruct((2 * _N, 8), jnp.float32),
    mesh=_mesh,
    compiler_params=pltpu.CompilerParams(use_tc_tiling_on_sc=False, needs_layout_passes=False),
    scratch_types=[
        pltpu.VMEM((_K,), jnp.int32),
        pltpu.VMEM((_K,), jnp.int32),
        pltpu.VMEM((_K, 8), jnp.float32),
        pltpu.VMEM((_K, 8), jnp.float32),
        pltpu.VMEM((4 * _K,), jnp.float32),
        pltpu.VMEM((_K, 8), jnp.float32),
        pltpu.VMEM_SHARED((_N, 8), jnp.float32),
        pltpu.SemaphoreType.DMA,
    ],
)(_edge_ab_body)


def _edge_div_body(tbl_hbm, ei_hbm, z1_hbm, out_hbm,
                   rows_v, cols_v, arow_v, acol_v, contrib_v, acc_sh, sem):
    cid = lax.axis_index("c")
    sid = lax.axis_index("s")
    wid = cid * _NS + sid

    astart = pl.multiple_of(((sid * _RPT) >> 3) << 3, 8)
    pltpu.sync_copy(z1_hbm.at[pl.ds(0, _ASTRIPE)],
                    acc_sh.at[pl.ds(astart, _ASTRIPE)])
    plsc.subcore_barrier()

    lane = lax.iota(jnp.int32, 16)

    def chunk(i, carry):
        base = wid * _EPW + i * _K
        c1 = pltpu.async_copy(ei_hbm.at[pl.ds(base, _K)], rows_v, sem)
        c2 = pltpu.async_copy(ei_hbm.at[pl.ds(_E + base, _K)], cols_v, sem)
        c1.wait()
        c2.wait()
        g1 = pltpu.async_copy(tbl_hbm.at[rows_v], arow_v, sem)
        g2 = pltpu.async_copy(tbl_hbm.at[cols_v], acol_v, sem)
        g1.wait()
        g2.wait()

        def grp(g, carry2):
            idx = g * 16 + lane

            def ld(ref, c):
                return plsc.load_gather(ref, [idx, _splat_i32(c)])

            nxr = ld(arow_v, 0)
            nyr = ld(arow_v, 1)
            nzr = ld(arow_v, 2)
            pxr = ld(arow_v, 3)
            pyr = ld(arow_v, 4)
            pzr = ld(arow_v, 5)
            nxc = ld(acol_v, 0)
            nyc = ld(acol_v, 1)
            nzc = ld(acol_v, 2)
            pxc = ld(acol_v, 3)
            pyc = ld(acol_v, 4)
            pzc = ld(acol_v, 5)

            rx = pxc - pxr
            ry = pyc - pyr
            rz = pzc - pzr
            sb = rx * rx + ry * ry + rz * rz
            ii = plsc.bitcast(sb, jnp.int32)
            ii = jnp.int32(0x5F3759DF) - (ii >> 1)
            y = plsc.bitcast(ii, jnp.float32)
            y = y * (1.5 - 0.5 * sb * y * y)
            y = y * (1.5 - 0.5 * sb * y * y)
            dist = sb * y
            den = jnp.maximum(dist, 1e-8) + 1e-8
            num = ((nxc - nxr) * rx + (nyc - nyr) * ry + (nzc - nzr) * rz)
            contrib_v[pl.ds(g * 16, 16)] = num / den
            return carry2

        lax.fori_loop(0, _G, grp, 0)
        pltpu.sync_copy(contrib_v, acc_sh.at[rows_v], add=True)
        return carry

    lax.fori_loop(0, _NCHUNK, chunk, 0)
    plsc.subcore_barrier()
    astart2 = pl.multiple_of(((sid * _RPT) >> 3) << 3, 8)
    pltpu.sync_copy(acc_sh.at[pl.ds(astart2, _ASTRIPE)],
                    out_hbm.at[pl.ds(cid * _N + astart2, _ASTRIPE)])


_edge_div = functools.partial(
    pl.kernel,
    out_type=jax.ShapeDtypeStruct((2 * _N,), jnp.float32),
    mesh=_mesh,
    compiler_params=pltpu.CompilerParams(use_tc_tiling_on_sc=False, needs_layout_passes=False),
    scratch_types=[
        pltpu.VMEM((_K,), jnp.int32),
        pltpu.VMEM((_K,), jnp.int32),
        pltpu.VMEM((_K, 8), jnp.float32),
        pltpu.VMEM((_K, 8), jnp.float32),
        pltpu.VMEM((_K,), jnp.float32),
        pltpu.VMEM_SHARED((_N,), jnp.float32),
        pltpu.SemaphoreType.DMA,
    ],
)(_edge_div_body)


def _node_body(a0, a1, tbl, tblc, dinv, scal):
    i = pl.program_id(0)
    a = a0[...] + a1[...]
    deg = jnp.maximum(a[:, 6:7], 1.0)
    ga = a[:, 0:3] / deg
    gb = a[:, 3:6] / deg
    gan = jnp.sqrt(jnp.sum(ga * ga, axis=1, keepdims=True))
    gbn = jnp.maximum(jnp.sqrt(jnp.sum(gb * gb, axis=1, keepdims=True)), 1e-8)
    normal = gb / gbn
    phi = tbl[:, 0:1]
    pos = tbl[:, 1:4]
    phic = tbl[:, 4:5]
    tblc[...] = jnp.concatenate(
        [normal, pos, jnp.zeros_like(normal[:, 0:2])], axis=1)
    dinv[...] = 1.0 / deg
    mask = jnp.abs(phi) < 0.3
    isum = jnp.sum(jnp.where(mask, (gan - 1.0) ** 2, 0.0))
    icnt = jnp.sum(mask.astype(jnp.float32))
    vc = jnp.sum((phic > 0).astype(jnp.float32))
    vn = jnp.sum((phi > 0).astype(jnp.float32))
    lanes = lax.broadcasted_iota(jnp.int32, (1, 128), 1)
    part = (jnp.where(lanes == 0, isum, 0.0)
            + jnp.where(lanes == 1, icnt, 0.0)
            + jnp.where(lanes == 2, vc, 0.0)
            + jnp.where(lanes == 3, vn, 0.0))

    @pl.when(i == 0)
    def _():
        scal[...] = jnp.zeros_like(scal)

    scal[...] += part


def _loss_body(c0, c1, dinv, scal, tot):
    i = pl.program_id(0)
    curv = (c0[...] + c1[...]) * dinv[...]
    part = jnp.sum(curv * curv).reshape(1, 1)

    @pl.when(i == 0)
    def _():
        tot[...] = jnp.zeros_like(tot)

    tot[...] += part

    @pl.when(i == pl.num_programs(0) - 1)
    def _():
        s = tot[...][0, 0]
        sv = scal[...]
        isum = sv[0, 0]
        icnt = sv[0, 1]
        vc = sv[0, 2]
        vn = sv[0, 3]
        interface = jnp.where(icnt > 0, isum / jnp.maximum(icnt, 1.0), 0.0)
        vol = ((vn - vc) / (vc + 1e-8)) ** 2
        total = interface + 0.1 * (s / _N) + 0.01 * vol
        tot[...] = total.reshape(1, 1)


def kernel(pred, target, x, pos, edge_index, edge_attr):
    phi_c = x[:, 8]
    phi = phi_c + pred[:, 5]
    zeros3 = jnp.zeros((_N, 3), jnp.float32)
    tbl1 = jnp.concatenate(
        [phi[:, None], pos, phi_c[:, None], zeros3], axis=1)
    ei1d = edge_index.reshape(2 * _E)
    ea1d = edge_attr.reshape(4 * _E)
    z8 = jnp.zeros((_RPT, 8), jnp.float32)
    z1 = jnp.zeros((_ASTRIPE,), jnp.float32)

    acc2 = _edge_ab(tbl1, ei1d, ea1d, z8)

    tblc, dinv, scal = pl.pallas_call(
        _node_body,
        grid=(_GRID,),
        in_specs=[
            pl.BlockSpec((_R, 8), lambda i: (i, 0)),
            pl.BlockSpec((_R, 8), lambda i: (i + _GRID, 0)),
            pl.BlockSpec((_R, 8), lambda i: (i, 0)),
        ],
        out_specs=[
            pl.BlockSpec((_R, 8), lambda i: (i, 0)),
            pl.BlockSpec((_R, 1), lambda i: (i, 0)),
            pl.BlockSpec((1, 128), lambda i: (0, 0)),
        ],
        out_shape=[
            jax.ShapeDtypeStruct((_N, 8), jnp.float32),
            jax.ShapeDtypeStruct((_N, 1), jnp.float32),
            jax.ShapeDtypeStruct((1, 128), jnp.float32),
        ],
    )(acc2, acc2, tbl1)

    curv2 = _edge_div(tblc, ei1d, z1)
    curv2 = curv2.reshape(2 * _N, 1)

    tot = pl.pallas_call(
        _loss_body,
        grid=(_GRID,),
        in_specs=[
            pl.BlockSpec((_R, 1), lambda i: (i, 0)),
            pl.BlockSpec((_R, 1), lambda i: (i + _GRID, 0)),
            pl.BlockSpec((_R, 1), lambda i: (i, 0)),
            pl.BlockSpec((1, 128), lambda i: (0, 0)),
        ],
        out_specs=pl.BlockSpec((1, 1), lambda i: (0, 0)),
        out_shape=jax.ShapeDtypeStruct((1, 1), jnp.float32),
    )(curv2, curv2, dinv, scal)

    return tot[0, 0]
